# Initial kernel scaffold; baseline (speedup 1.0000x reference)
import functools

import jax
import jax.numpy as jnp
from jax import lax
from jax.experimental import pallas as pl
from jax.experimental.pallas import tpu as pltpu
from jax.experimental.pallas import tpu_sc as plsc

N_NODES = 10000
N_EDGES = 320000
D_IN = 128
HIDDEN = 128
N_CLASSES = 40

N_CORES = 2
N_SUBCORES = 16
N_WORKERS = N_CORES * N_SUBCORES

N_PAD = 10240
CHUNK = 128
CHUNKS_PER_W = 79
E_PAD = N_WORKERS * CHUNKS_PER_W * CHUNK
ROWS_PER_TILE = N_PAD // N_SUBCORES


def _make_agg(D, use_table):
    mesh = plsc.VectorSubcoreMesh(core_axis_name="c", subcore_axis_name="s")
    scratch = [
        pltpu.VMEM((CHUNK,), jnp.int32),
        pltpu.VMEM((CHUNK,), jnp.int32),
        pltpu.VMEM((CHUNK, D), jnp.float32),
        pltpu.VMEM_SHARED((N_PAD, D), jnp.float32),
        pltpu.SemaphoreType.DMA,
    ]

    def body(*refs):
        if use_table:
            (table_hbm, src_hbm, dst_hbm, out_hbm,
             src_v, dst_v, rows_v, acc_sh, sem) = refs
        else:
            src_hbm, dst_hbm, out_hbm, src_v, dst_v, rows_v, acc_sh, sem = refs
            table_hbm = None
        c = lax.axis_index("c")
        s = lax.axis_index("s")
        wid = c * N_SUBCORES + s

        zero16 = jnp.zeros((16,), jnp.float32)
        cols16 = D // 16

        def zbody(i, carry):
            rows_v[i // cols16, pl.ds((i % cols16) * 16, 16)] = zero16
            return carry

        lax.fori_loop(0, CHUNK * cols16, zbody, 0)
        for j in range(ROWS_PER_TILE // CHUNK):
            pltpu.sync_copy(
                rows_v,
                acc_sh.at[pl.ds(s * ROWS_PER_TILE + j * CHUNK, CHUNK)])
        if not use_table:
            one16 = jnp.ones((16,), jnp.float32)

            def obody(i, carry):
                rows_v[i // cols16, pl.ds((i % cols16) * 16, 16)] = one16
                return carry

            lax.fori_loop(0, CHUNK * cols16, obody, 0)
        plsc.subcore_barrier()

        base = wid * (CHUNKS_PER_W * CHUNK)

        def ebody(i, carry):
            off = base + i * CHUNK
            pltpu.sync_copy(dst_hbm.at[pl.ds(off, CHUNK)], dst_v)
            if use_table:
                pltpu.sync_copy(src_hbm.at[pl.ds(off, CHUNK)], src_v)
                pltpu.async_copy(table_hbm.at[src_v], rows_v, sem).wait()
            pltpu.sync_copy(rows_v, acc_sh.at[dst_v], add=True)
            return carry

        lax.fori_loop(0, CHUNKS_PER_W, ebody, 0)
        plsc.subcore_barrier()
        pltpu.sync_copy(
            acc_sh.at[pl.ds(s * ROWS_PER_TILE, ROWS_PER_TILE)],
            out_hbm.at[c, pl.ds(s * ROWS_PER_TILE, ROWS_PER_TILE)])

    return functools.partial(
        pl.kernel, mesh=mesh,
        out_type=jax.ShapeDtypeStruct((N_CORES, N_PAD, D), jnp.float32),
        scratch_types=scratch)(body)


_agg_deg = _make_agg(16, use_table=False)
_agg_h1 = _make_agg(HIDDEN, use_table=True)
_agg_h2 = _make_agg(48, use_table=True)


def _tc1_body(degp_ref, x_ref, w1_ref, t1_ref, dinv_ref):
    deg = degp_ref[0, :, 0:1] + degp_ref[1, :, 0:1] + 1.0
    dinv = lax.rsqrt(deg)
    h = jnp.dot(x_ref[...], w1_ref[...], preferred_element_type=jnp.float32)
    t1_ref[...] = h * dinv
    dinv_ref[...] = dinv


def _tc2_body(p_ref, t1_ref, dinv_ref, b1_ref, w2_ref, t2_ref):
    agg = p_ref[0] + p_ref[1] + t1_ref[...]
    out1 = jnp.maximum(agg * dinv_ref[...] + b1_ref[...], 0.0)
    h2 = jnp.dot(out1, w2_ref[...], preferred_element_type=jnp.float32)
    t2_ref[...] = h2 * dinv_ref[...]


def _tc3_body(p_ref, t2_ref, dinv_ref, b2_ref, out_ref):
    o = (p_ref[0] + p_ref[1] + t2_ref[...]) * dinv_ref[...] + b2_ref[...]
    col = lax.broadcasted_iota(jnp.int32, (N_PAD, 48), 1)
    mask = col < N_CLASSES
    om = jnp.where(mask, o, -jnp.inf)
    m = jnp.max(om, axis=1, keepdims=True)
    lse = m + jnp.log(jnp.sum(jnp.where(mask, jnp.exp(om - m), 0.0),
                              axis=1, keepdims=True))
    out_ref[...] = o - lse


def kernel(x, edge_index, W1, b1, W2, b2):
    src = edge_index[0].astype(jnp.int32)
    dst = edge_index[1].astype(jnp.int32)
    pad = jnp.full((E_PAD - N_EDGES,), N_NODES, jnp.int32)
    src_p = jnp.concatenate([src, pad])
    dst_p = jnp.concatenate([dst, pad])
    x_pad = jnp.zeros((N_PAD, D_IN), jnp.float32).at[:N_NODES].set(x)
    w2_pad = jnp.zeros((HIDDEN, 48), jnp.float32).at[:, :N_CLASSES].set(W2)
    b2_pad = jnp.zeros((48,), jnp.float32).at[:N_CLASSES].set(b2)

    degp = _agg_deg(src_p, dst_p)

    t1, dinv = pl.pallas_call(
        _tc1_body,
        out_shape=(jax.ShapeDtypeStruct((N_PAD, HIDDEN), jnp.float32),
                   jax.ShapeDtypeStruct((N_PAD, 1), jnp.float32)),
    )(degp, x_pad, W1)

    agg1 = _agg_h1(t1, src_p, dst_p)

    t2 = pl.pallas_call(
        _tc2_body,
        out_shape=jax.ShapeDtypeStruct((N_PAD, 48), jnp.float32),
    )(agg1, t1, dinv, b1, w2_pad)

    agg2 = _agg_h2(t2, src_p, dst_p)

    out_pad = pl.pallas_call(
        _tc3_body,
        out_shape=jax.ShapeDtypeStruct((N_PAD, 48), jnp.float32),
    )(agg2, t2, dinv, b2_pad)

    return out_pad[:N_NODES, :N_CLASSES]


# speedup vs baseline: 13.0843x; 13.0843x over previous
import functools

import jax
import jax.numpy as jnp
from jax import lax
from jax.experimental import pallas as pl
from jax.experimental.pallas import tpu as pltpu
from jax.experimental.pallas import tpu_sc as plsc

N_NODES = 10000
N_EDGES = 320000
D_IN = 128
HIDDEN = 128
N_CLASSES = 40

N_CORES = 2
N_SUBCORES = 16
N_WORKERS = N_CORES * N_SUBCORES

N_PAD = 10240
CHUNK = 128
CHUNKS_PER_W = 79
E_PAD = N_WORKERS * CHUNKS_PER_W * CHUNK
ROWS_PER_TILE = N_PAD // N_SUBCORES


def _make_agg(D, use_table):
    mesh = plsc.VectorSubcoreMesh(core_axis_name="c", subcore_axis_name="s")
    scratch = [
        pltpu.VMEM((CHUNK,), jnp.int32),
        pltpu.VMEM((CHUNK,), jnp.int32),
        pltpu.VMEM((CHUNK, D), jnp.float32),
        pltpu.VMEM_SHARED((N_PAD, D), jnp.float32),
        pltpu.SemaphoreType.DMA,
    ]

    def body(*refs):
        if use_table:
            (table_hbm, src_hbm, dst_hbm, out_hbm,
             src_v, dst_v, rows_v, acc_sh, sem) = refs
        else:
            src_hbm, dst_hbm, out_hbm, src_v, dst_v, rows_v, acc_sh, sem = refs
            table_hbm = None
        c = lax.axis_index("c")
        s = lax.axis_index("s")
        wid = c * N_SUBCORES + s

        zero16 = jnp.zeros((16,), jnp.float32)
        cols16 = D // 16

        def zbody(i, carry):
            rows_v[i // cols16, pl.ds((i % cols16) * 16, 16)] = zero16
            return carry

        lax.fori_loop(0, CHUNK * cols16, zbody, 0)
        for j in range(ROWS_PER_TILE // CHUNK):
            pltpu.sync_copy(
                rows_v,
                acc_sh.at[pl.ds(s * ROWS_PER_TILE + j * CHUNK, CHUNK)])
        if not use_table:
            one16 = jnp.ones((16,), jnp.float32)

            def obody(i, carry):
                rows_v[i // cols16, pl.ds((i % cols16) * 16, 16)] = one16
                return carry

            lax.fori_loop(0, CHUNK * cols16, obody, 0)
        plsc.subcore_barrier()

        base = wid * (CHUNKS_PER_W * CHUNK)

        def ebody(i, carry):
            off = base + i * CHUNK
            pltpu.sync_copy(dst_hbm.at[pl.ds(off, CHUNK)], dst_v)
            if use_table:
                pltpu.sync_copy(src_hbm.at[pl.ds(off, CHUNK)], src_v)
                pltpu.async_copy(table_hbm.at[src_v], rows_v, sem).wait()
            pltpu.sync_copy(rows_v, acc_sh.at[dst_v], add=True)
            return carry

        lax.fori_loop(0, CHUNKS_PER_W, ebody, 0)
        plsc.subcore_barrier()
        pltpu.sync_copy(
            acc_sh.at[pl.ds(s * ROWS_PER_TILE, ROWS_PER_TILE)],
            out_hbm.at[c, pl.ds(s * ROWS_PER_TILE, ROWS_PER_TILE)])

    return functools.partial(
        pl.kernel, mesh=mesh,
        out_type=jax.ShapeDtypeStruct((N_CORES, N_PAD, D), jnp.float32),
        compiler_params=pltpu.CompilerParams(use_tc_tiling_on_sc=(D % 128 == 0)),
        scratch_types=scratch)(body)


_agg_deg = _make_agg(16, use_table=False)
_agg_h1 = _make_agg(HIDDEN, use_table=True)
_agg_h2 = _make_agg(48, use_table=True)


def _tc1_body(degp_ref, x_ref, w1_ref, t1_ref, dinv_ref):
    deg = degp_ref[0, :, 0:1] + degp_ref[1, :, 0:1] + 1.0
    dinv = lax.rsqrt(deg)
    h = jnp.dot(x_ref[...], w1_ref[...], preferred_element_type=jnp.float32)
    t1_ref[...] = h * dinv
    dinv_ref[...] = dinv


def _tc2_body(p_ref, t1_ref, dinv_ref, b1_ref, w2_ref, t2_ref):
    agg = p_ref[0] + p_ref[1] + t1_ref[...]
    out1 = jnp.maximum(agg * dinv_ref[...] + b1_ref[...], 0.0)
    h2 = jnp.dot(out1, w2_ref[...], preferred_element_type=jnp.float32)
    t2_ref[...] = h2 * dinv_ref[...]


def _tc3_body(p_ref, t2_ref, dinv_ref, b2_ref, out_ref):
    o = (p_ref[0] + p_ref[1] + t2_ref[...]) * dinv_ref[...] + b2_ref[...]
    col = lax.broadcasted_iota(jnp.int32, (N_PAD, 48), 1)
    mask = col < N_CLASSES
    om = jnp.where(mask, o, -jnp.inf)
    m = jnp.max(om, axis=1, keepdims=True)
    lse = m + jnp.log(jnp.sum(jnp.where(mask, jnp.exp(om - m), 0.0),
                              axis=1, keepdims=True))
    out_ref[...] = o - lse


def kernel(x, edge_index, W1, b1, W2, b2):
    src = edge_index[0].astype(jnp.int32)
    dst = edge_index[1].astype(jnp.int32)
    pad = jnp.full((E_PAD - N_EDGES,), N_NODES, jnp.int32)
    src_p = jnp.concatenate([src, pad])
    dst_p = jnp.concatenate([dst, pad])
    x_pad = jnp.zeros((N_PAD, D_IN), jnp.float32).at[:N_NODES].set(x)
    w2_pad = jnp.zeros((HIDDEN, 48), jnp.float32).at[:, :N_CLASSES].set(W2)
    b2_pad = jnp.zeros((48,), jnp.float32).at[:N_CLASSES].set(b2)

    degp = _agg_deg(src_p, dst_p)

    t1, dinv = pl.pallas_call(
        _tc1_body,
        out_shape=(jax.ShapeDtypeStruct((N_PAD, HIDDEN), jnp.float32),
                   jax.ShapeDtypeStruct((N_PAD, 1), jnp.float32)),
    )(degp, x_pad, W1)

    agg1 = _agg_h1(t1, src_p, dst_p)

    t2 = pl.pallas_call(
        _tc2_body,
        out_shape=jax.ShapeDtypeStruct((N_PAD, 48), jnp.float32),
    )(agg1, t1, dinv, b1, w2_pad)

    agg2 = _agg_h2(t2, src_p, dst_p)

    out_pad = pl.pallas_call(
        _tc3_body,
        out_shape=jax.ShapeDtypeStruct((N_PAD, 48), jnp.float32),
    )(agg2, t2, dinv, b2_pad)

    return out_pad[:N_NODES, :N_CLASSES]


# speedup vs baseline: 14.2556x; 1.0895x over previous
import functools

import jax
import jax.numpy as jnp
from jax import lax
from jax.experimental import pallas as pl
from jax.experimental.pallas import tpu as pltpu
from jax.experimental.pallas import tpu_sc as plsc

N_NODES = 10000
N_EDGES = 320000
D_IN = 128
HIDDEN = 128
N_CLASSES = 40

N_CORES = 2
N_SUBCORES = 16
N_WORKERS = N_CORES * N_SUBCORES

N_PAD = 10240
CHUNK = 128
CHUNKS_PER_W = 80
E_PAD = N_WORKERS * CHUNKS_PER_W * CHUNK
ROWS_PER_TILE = N_PAD // N_SUBCORES


def _fill(ref, val16, cols16):
    def fbody(i, carry):
        ref[i // cols16, pl.ds((i % cols16) * 16, 16)] = val16
        return carry
    lax.fori_loop(0, CHUNK * cols16, fbody, 0)


def _zero_acc(rows0, acc_sh, s, cols16):
    _fill(rows0, jnp.zeros((16,), jnp.float32), cols16)
    for j in range(ROWS_PER_TILE // CHUNK):
        pltpu.sync_copy(
            rows0, acc_sh.at[pl.ds(s * ROWS_PER_TILE + j * CHUNK, CHUNK)])
    plsc.subcore_barrier()


def _writeback(acc_sh, out_hbm, c, s):
    plsc.subcore_barrier()
    pltpu.sync_copy(
        acc_sh.at[pl.ds(s * ROWS_PER_TILE, ROWS_PER_TILE)],
        out_hbm.at[c, pl.ds(s * ROWS_PER_TILE, ROWS_PER_TILE)])


def _make_agg(D):
    mesh = plsc.VectorSubcoreMesh(core_axis_name="c", subcore_axis_name="s")
    scratch = [
        pltpu.VMEM((4, 2, CHUNK), jnp.int32),
        pltpu.VMEM((CHUNK, D), jnp.float32),
        pltpu.VMEM((CHUNK, D), jnp.float32),
        pltpu.VMEM_SHARED((N_PAD, D), jnp.float32),
        pltpu.SemaphoreType.DMA,
        pltpu.SemaphoreType.DMA,
        pltpu.SemaphoreType.DMA,
        pltpu.SemaphoreType.DMA,
        pltpu.SemaphoreType.DMA,
        pltpu.SemaphoreType.DMA,
    ]

    def body(table_hbm, ei_hbm, out_hbm, ring, rows0, rows1, acc_sh,
             g0, g1, i0, i1, i2, i3):
        bufs = (rows0, rows1)
        gsems = (g0, g1)
        isems = (i0, i1, i2, i3)
        c = lax.axis_index("c")
        s = lax.axis_index("s")
        wid = c * N_SUBCORES + s
        n = CHUNKS_PER_W
        rbase = wid * CHUNKS_PER_W

        _zero_acc(rows0, acc_sh, s, D // 16)

        def ifire(i, slot):
            pltpu.async_copy(ei_hbm.at[rbase + i], ring.at[slot],
                             isems[slot])

        def iwait(i, slot):
            pltpu.make_async_copy(ei_hbm.at[rbase + i], ring.at[slot],
                                  isems[slot]).wait()

        def gfire(slot, b):
            pltpu.async_copy(table_hbm.at[ring.at[slot, 0]], bufs[b],
                             gsems[b])

        def gwait(slot, b):
            pltpu.make_async_copy(table_hbm.at[ring.at[slot, 0]], bufs[b],
                                  gsems[b]).wait()

        for slot in range(4):
            ifire(slot, slot)
        iwait(0, 0)
        gfire(0, 0)

        def outer(g, carry):
            for b in range(4):
                i = g * 4 + b

                @pl.when(i + 1 < n)
                def _():
                    iwait(i + 1, (b + 1) % 4)
                    gfire((b + 1) % 4, (b + 1) % 2)

                gwait(b, b % 2)
                pltpu.sync_copy(bufs[b % 2], acc_sh.at[ring.at[b, 1]],
                                add=True)

                @pl.when(i + 4 < n)
                def _():
                    ifire(i + 4, b)
            return carry

        lax.fori_loop(0, n // 4, outer, 0)
        _writeback(acc_sh, out_hbm, c, s)

    return functools.partial(
        pl.kernel, mesh=mesh,
        out_type=jax.ShapeDtypeStruct((N_CORES, N_PAD, D), jnp.float32),
        compiler_params=pltpu.CompilerParams(use_tc_tiling_on_sc=(D % 128 == 0)),
        scratch_types=scratch)(body)


def _make_deg():
    D = 16
    mesh = plsc.VectorSubcoreMesh(core_axis_name="c", subcore_axis_name="s")
    scratch = [
        pltpu.VMEM((CHUNKS_PER_W, CHUNK), jnp.int32),
        pltpu.VMEM((CHUNK, D), jnp.float32),
        pltpu.VMEM_SHARED((N_PAD, D), jnp.float32),
    ]

    def body(dst_hbm, out_hbm, dst_v, rows0, acc_sh):
        c = lax.axis_index("c")
        s = lax.axis_index("s")
        wid = c * N_SUBCORES + s

        _zero_acc(rows0, acc_sh, s, D // 16)
        pltpu.sync_copy(dst_hbm.at[pl.ds(wid * CHUNKS_PER_W, CHUNKS_PER_W)],
                        dst_v)
        _fill(rows0, jnp.ones((16,), jnp.float32), D // 16)

        def ebody(i, carry):
            pltpu.sync_copy(rows0, acc_sh.at[dst_v.at[i]], add=True)
            return carry

        lax.fori_loop(0, CHUNKS_PER_W, ebody, 0)
        _writeback(acc_sh, out_hbm, c, s)

    return functools.partial(
        pl.kernel, mesh=mesh,
        out_type=jax.ShapeDtypeStruct((N_CORES, N_PAD, D), jnp.float32),
        scratch_types=scratch)(body)


_agg_deg = _make_deg()
_agg_h1 = _make_agg(HIDDEN)
_agg_h2 = _make_agg(48)


def _tc1_body(degp_ref, x_ref, w1_ref, t1_ref, dinv_ref):
    deg = degp_ref[0, :, 0:1] + degp_ref[1, :, 0:1] + 1.0
    dinv = lax.rsqrt(deg)
    h = jnp.dot(x_ref[...], w1_ref[...], preferred_element_type=jnp.float32)
    t1_ref[...] = h * dinv
    dinv_ref[...] = dinv


def _tc2_body(p_ref, t1_ref, dinv_ref, b1_ref, w2_ref, t2_ref):
    agg = p_ref[0] + p_ref[1] + t1_ref[...]
    out1 = jnp.maximum(agg * dinv_ref[...] + b1_ref[...], 0.0)
    h2 = jnp.dot(out1, w2_ref[...], preferred_element_type=jnp.float32)
    t2_ref[...] = h2 * dinv_ref[...]


def _tc3_body(p_ref, t2_ref, dinv_ref, b2_ref, out_ref):
    o = (p_ref[0] + p_ref[1] + t2_ref[...]) * dinv_ref[...] + b2_ref[...]
    col = lax.broadcasted_iota(jnp.int32, (N_PAD, 48), 1)
    mask = col < N_CLASSES
    om = jnp.where(mask, o, -jnp.inf)
    m = jnp.max(om, axis=1, keepdims=True)
    lse = m + jnp.log(jnp.sum(jnp.where(mask, jnp.exp(om - m), 0.0),
                              axis=1, keepdims=True))
    out_ref[...] = o - lse


def kernel(x, edge_index, W1, b1, W2, b2):
    src = edge_index[0].astype(jnp.int32)
    dst = edge_index[1].astype(jnp.int32)
    pad = jnp.full((E_PAD - N_EDGES,), N_NODES, jnp.int32)
    src_p = jnp.concatenate([src, pad]).reshape(E_PAD // CHUNK, CHUNK)
    dst_p = jnp.concatenate([dst, pad]).reshape(E_PAD // CHUNK, CHUNK)
    ei_p = jnp.stack([src_p, dst_p], axis=1)
    x_pad = jnp.zeros((N_PAD, D_IN), jnp.float32).at[:N_NODES].set(x)
    w2_pad = jnp.zeros((HIDDEN, 48), jnp.float32).at[:, :N_CLASSES].set(W2)
    b2_pad = jnp.zeros((48,), jnp.float32).at[:N_CLASSES].set(b2)

    degp = _agg_deg(dst_p)

    t1, dinv = pl.pallas_call(
        _tc1_body,
        out_shape=(jax.ShapeDtypeStruct((N_PAD, HIDDEN), jnp.float32),
                   jax.ShapeDtypeStruct((N_PAD, 1), jnp.float32)),
    )(degp, x_pad, W1)

    agg1 = _agg_h1(t1, ei_p)

    t2 = pl.pallas_call(
        _tc2_body,
        out_shape=jax.ShapeDtypeStruct((N_PAD, 48), jnp.float32),
    )(agg1, t1, dinv, b1, w2_pad)

    agg2 = _agg_h2(t2, ei_p)

    out_pad = pl.pallas_call(
        _tc3_body,
        out_shape=jax.ShapeDtypeStruct((N_PAD, 48), jnp.float32),
    )(agg2, t2, dinv, b2_pad)

    return out_pad[:N_NODES, :N_CLASSES]
